# baseline (speedup 1.0000x reference)
import functools

import jax
import jax.numpy as jnp
from jax import lax
from jax.experimental import pallas as pl
from jax.experimental.pallas import tpu as pltpu
from jax.experimental.pallas import tpu_sc as plsc

VOCAB = 1000000
EMBED_DIM = 16
BATCH = 16384
HIST_LEN = 50
B = BATCH * HIST_LEN

NC = 2
NS = 16
NW = NC * NS
L = 16

TR = 1000
TSTEPS = VOCAB // TR

BATCH_PER_W = BATCH // NW
GB = 16
GCH = GB * HIST_LEN
NGC = BATCH_PER_W // GB

PLANE_BLK = 125000


@functools.partial(
    pl.pallas_call,
    out_shape=jax.ShapeDtypeStruct((VOCAB * EMBED_DIM,), jnp.float32),
    grid=(EMBED_DIM, VOCAB // PLANE_BLK),
    in_specs=[pl.BlockSpec((1, PLANE_BLK), lambda e, c: (e, c))],
    out_specs=pl.BlockSpec((PLANE_BLK,),
                           lambda e, c: (e * (VOCAB // PLANE_BLK) + c,)),
)
def _flatten_planes(tab_ref, out_ref):
    out_ref[...] = tab_ref[0, :]


@functools.partial(
    pl.kernel,
    out_type=jax.ShapeDtypeStruct((HIST_LEN, EMBED_DIM, BATCH), jnp.float32),
    mesh=plsc.VectorSubcoreMesh(core_axis_name="c", subcore_axis_name="s"),
    scratch_types=dict(
        tabA=pltpu.HBM((VOCAB, EMBED_DIM), jnp.float32),
        tabB=pltpu.HBM((VOCAB, EMBED_DIM), jnp.float32),
        colv=pltpu.VMEM((EMBED_DIM, TR), jnp.float32),
        aos=pltpu.VMEM((TR, EMBED_DIM), jnp.float32),
        idxv=pltpu.VMEM((GCH,), jnp.int32),
        rows=pltpu.VMEM((GCH, EMBED_DIM), jnp.float32),
        soa=pltpu.VMEM((HIST_LEN, EMBED_DIM, GB), jnp.float32),
        gsem=pltpu.SemaphoreType.DMA,
    ),
    compiler_params=pltpu.CompilerParams(use_tc_tiling_on_sc=False,
                                         needs_layout_passes=False),
)
def _gather_kernel(idx_hbm, flat_hbm, out_hbm, *, tabA, tabB, colv, aos,
                   idxv, rows, soa, gsem):
    c = lax.axis_index("c")
    s = lax.axis_index("s")
    wid = s * NC + c
    base = wid * BATCH_PER_W * HIST_LEN
    bbase = wid * BATCH_PER_W

    lanes = lax.iota(jnp.int32, L)
    ones = jnp.ones((L,), jnp.int32)

    def transpose_phase(tab):
        n_full = TSTEPS // NS
        n_rem = TSTEPS % NS
        n_t = jnp.where(s < n_rem, n_full + 1, n_full) if n_rem else n_full

        def step(t, carry):
            g = t * NS + s
            r0 = g * TR
            for e in range(EMBED_DIM):
                pltpu.sync_copy(flat_hbm.at[pl.ds(e * VOCAB + r0, TR)],
                                colv.at[e])

            def interleave(r8, carry2):
                for j in range(8):
                    r = r8 * 8 + j
                    rv = r * ones
                    vals = plsc.load_gather(colv, [lanes, rv])
                    plsc.store_scatter(aos, [rv, lanes], vals)
                return carry2

            lax.fori_loop(0, TR // 8, interleave, 0)
            pltpu.sync_copy(aos, tab.at[pl.ds(r0, TR), :])
            return carry

        lax.fori_loop(0, n_t, step, 0)

    def gather_phase(tab):
        def step(k, carry):
            off = base + k * GCH
            pltpu.sync_copy(idx_hbm.at[pl.ds(off, GCH)], idxv)
            pltpu.async_copy(tab.at[idxv], rows, gsem).wait()

            def repack(h, carry2):
                h16 = h * ones
                for bb in range(GB):
                    vals = rows[bb * HIST_LEN + h, :]
                    plsc.store_scatter(soa, [h16, lanes, bb * ones], vals)
                return carry2

            lax.fori_loop(0, HIST_LEN, repack, 0)
            bg0 = bbase + k * GB
            pltpu.sync_copy(soa, out_hbm.at[:, :, pl.ds(bg0, GB)])
            return carry

        lax.fori_loop(0, NGC, step, 0)

    @pl.when(c == 0)
    def _():
        transpose_phase(tabA)

    @pl.when(c == 1)
    def _():
        transpose_phase(tabB)

    plsc.subcore_barrier()

    @pl.when(c == 0)
    def _():
        gather_phase(tabA)

    @pl.when(c == 1)
    def _():
        gather_phase(tabB)


def kernel(inputs, embeddings):
    idx_flat = inputs.reshape(B)
    flat_table = embeddings.T.reshape(VOCAB * EMBED_DIM)
    out_t = _gather_kernel(idx_flat, flat_table)
    return jnp.transpose(out_t, (2, 0, 1))


# speedup vs baseline: 3.4952x; 3.4952x over previous
import functools

import jax
import jax.numpy as jnp
from jax import lax
from jax.experimental import pallas as pl
from jax.experimental.pallas import tpu as pltpu
from jax.experimental.pallas import tpu_sc as plsc

VOCAB = 1000000
EMBED_DIM = 16
BATCH = 16384
HIST_LEN = 50
HPAD = 56
B = BATCH * HIST_LEN

NC = 2
NS = 16
NW = NC * NS
BATCH_PER_W = BATCH // NW
B_PER_W = B // NW
CHUNK_BATCH = 32
CHUNK = CHUNK_BATCH * HIST_LEN
NCHUNK = BATCH_PER_W // CHUNK_BATCH
NB = 3


@functools.partial(
    pl.kernel,
    out_type=jax.ShapeDtypeStruct((BATCH * HPAD, 128), jnp.float32),
    mesh=plsc.VectorSubcoreMesh(core_axis_name="c", subcore_axis_name="s"),
    scratch_types=(
        [pltpu.VMEM((CHUNK,), jnp.int32) for _ in range(NB)]
        + [pltpu.VMEM((CHUNK, EMBED_DIM), jnp.float32) for _ in range(NB)]
        + [pltpu.SemaphoreType.DMA for _ in range(2 * NB)]
    ),
    compiler_params=pltpu.CompilerParams(use_tc_tiling_on_sc=False),
)
def _gather_kernel(idx_hbm, table_hbm, out_hbm, *scratch):
    idx_v = scratch[0:NB]
    rows_v = scratch[NB:2 * NB]
    gsem = scratch[2 * NB:3 * NB]
    osem = scratch[3 * NB:4 * NB]

    wid = lax.axis_index("s") * NC + lax.axis_index("c")
    base = wid * B_PER_W
    bbase = wid * BATCH_PER_W

    def load_idx(i):
        off = base + i * CHUNK
        pltpu.sync_copy(idx_hbm.at[pl.ds(off, CHUNK)], idx_v[i % NB])

    def start_gather(i):
        b = i % NB
        pltpu.async_copy(table_hbm.at[idx_v[b]], rows_v[b], gsem[b])

    def wait_gather(i):
        b = i % NB
        pltpu.make_async_copy(table_hbm.at[idx_v[b]], rows_v[b],
                              gsem[b]).wait()

    def out_descrs(i):
        b = i % NB
        for j in range(CHUNK_BATCH):
            bg = bbase + i * CHUNK_BATCH + j
            yield pltpu.make_async_copy(
                rows_v[b].at[pl.ds(j * HIST_LEN, HIST_LEN), :],
                out_hbm.at[pl.ds(bg * HPAD, HIST_LEN), pl.ds(0, EMBED_DIM)],
                osem[b])

    def start_out(i):
        for d in out_descrs(i):
            d.start()

    def wait_out(i):
        for d in out_descrs(i):
            d.wait()

    load_idx(0)
    load_idx(1)
    start_gather(0)
    start_gather(1)
    for i in range(NCHUNK):
        wait_gather(i)
        start_out(i)
        if i + 2 < NCHUNK:
            load_idx(i + 2)
            if i >= 1:
                wait_out(i - 1)
            start_gather(i + 2)
    wait_out(NCHUNK - 2)
    wait_out(NCHUNK - 1)


def kernel(inputs, embeddings):
    idx_flat = inputs.reshape(B)
    staged = _gather_kernel(idx_flat, embeddings)
    return staged.reshape(BATCH, HPAD, 128)[:, :HIST_LEN, :EMBED_DIM]
